# Initial kernel scaffold; baseline (speedup 1.0000x reference)
import jax
import jax.numpy as jnp
from jax.experimental import pallas as pl


def kernel(x, edge_index, W1, b1, L1, Lb1, att1, W2, b2, L2, Lb2, att2, lin_w, lin_b):
    raise NotImplementedError("write your pallas kernel here")


# speedup vs baseline: 132.3829x; 132.3829x over previous
import functools
import jax
import jax.numpy as jnp
from jax import lax
from jax.experimental import pallas as pl
from jax.experimental.pallas import tpu as pltpu
from jax.experimental.pallas import tpu_sc as plsc

N = 10000
E = 160000
B = 8
F = 32
P = 8
HID = 64
CH = 128
EB = 128
EPAD = 163840
NBE = 40
NTILE = 16
NPAD = 10240
ROWS_T = NPAD // NTILE
NB = 2000
NBLK = N // NB

_mesh = plsc.VectorSubcoreMesh(core_axis_name="c", subcore_axis_name="s")


@functools.partial(
    pl.kernel,
    out_type=jax.ShapeDtypeStruct((2, NPAD), jnp.float32),
    mesh=_mesh,
    scratch_types=[
        pltpu.VMEM((NBE, EB), jnp.int32),
        pltpu.VMEM((EB,), jnp.float32),
        pltpu.VMEM_SHARED((NPAD,), jnp.float32),
    ],
)
def _deg_kernel(er, ones_h, zeros_h, deg_out, dst_v, ones_v, acc):
    cid = lax.axis_index("c")
    sid = lax.axis_index("s")
    pltpu.sync_copy(er.at[1, cid * NTILE + sid], dst_v)
    pltpu.sync_copy(ones_h, ones_v)
    pltpu.sync_copy(zeros_h, acc.at[pl.ds(sid * ROWS_T, ROWS_T)])
    plsc.subcore_barrier()

    def eb_body(eb, carry):
        pltpu.sync_copy(ones_v, acc.at[dst_v.at[eb]], add=True)
        return carry

    lax.fori_loop(0, NBE, eb_body, 0)
    plsc.subcore_barrier()
    pltpu.sync_copy(
        acc.at[pl.ds(sid * ROWS_T, ROWS_T)],
        deg_out.at[cid, pl.ds(sid * ROWS_T, ROWS_T)],
    )


def _make_prop(nchunk):
    nck = nchunk // 2

    @functools.partial(
        pl.kernel,
        out_type=jax.ShapeDtypeStruct((nchunk, NPAD, CH), jnp.float32),
        mesh=_mesh,
        scratch_types=[
            pltpu.VMEM((2 * NBE, EB), jnp.int32),
            pltpu.VMEM((2 * NBE, EB), jnp.int32),
            pltpu.VMEM((EB, CH), jnp.float32),
            pltpu.VMEM_SHARED((NPAD, CH), jnp.float32),
            pltpu.SemaphoreType.DMA,
        ],
    )
    def prop(xc, er, zeros_h, out, src_v, dst_v, rows_a, acc, sem_a):
        cid = lax.axis_index("c")
        sid = lax.axis_index("s")
        pltpu.sync_copy(er.at[0, sid], src_v.at[pl.ds(0, NBE)])
        pltpu.sync_copy(er.at[0, sid + NTILE], src_v.at[pl.ds(NBE, NBE)])
        pltpu.sync_copy(er.at[1, sid], dst_v.at[pl.ds(0, NBE)])
        pltpu.sync_copy(er.at[1, sid + NTILE], dst_v.at[pl.ds(NBE, NBE)])

        def chunk_body(ci, carry):
            chunk = cid * nck + ci
            pltpu.sync_copy(zeros_h, acc.at[pl.ds(sid * ROWS_T, ROWS_T)])
            plsc.subcore_barrier()

            def eb_body(eb, carry):
                pltpu.async_copy(xc.at[chunk].at[src_v.at[eb]], rows_a,
                                 sem_a).wait()
                pltpu.sync_copy(rows_a, acc.at[dst_v.at[eb]], add=True)
                return carry

            lax.fori_loop(0, 2 * NBE, eb_body, 0)
            plsc.subcore_barrier()
            pltpu.sync_copy(
                acc.at[pl.ds(sid * ROWS_T, ROWS_T)],
                out.at[chunk, pl.ds(sid * ROWS_T, ROWS_T)],
            )
            plsc.subcore_barrier()
            return carry

        lax.fori_loop(0, nck, chunk_body, 0)

    return prop


_prop16 = _make_prop(16)
_prop4 = _make_prop(4)


def _foldw_body(W1r, L1r, b1r, Lb1r, att1r, W2r, L2r, b2r, Lb2r, linwr,
                w0big, w2big, c04, c24, s0, s1, v0big, v2big, d02, d22,
                linbig):
    z3264 = jnp.zeros((F, HID), jnp.float32)
    z64 = jnp.zeros((HID, HID), jnp.float32)
    eye64 = jnp.eye(HID, dtype=jnp.float32)

    def fold1(g):
        Wp = jnp.dot(W1r[g], L1r[g, :HID, :],
                     preferred_element_type=jnp.float32)
        c = jnp.dot(b1r[g:g + 1, :], L1r[g, :HID, :],
                    preferred_element_type=jnp.float32) + Lb1r[g:g + 1, :]
        return Wp, c

    W0p, c0 = fold1(0)
    W2p, c2 = fold1(2)

    def bigdiag4(Wp):
        cols = []
        for j in range(4):
            blocks = [Wp if i == j else z3264 for i in range(4)]
            cols.append(jnp.concatenate(blocks, axis=0))
        return jnp.concatenate(cols, axis=1)

    w0big[...] = bigdiag4(W0p)
    w2big[...] = bigdiag4(W2p)
    c04[...] = jnp.concatenate([c0] * 4, axis=1)
    c24[...] = jnp.concatenate([c2] * 4, axis=1)

    probs = jax.nn.softmax(att1r[...], axis=-1)
    s0[...] = jnp.concatenate(
        [probs[0:1, p_:p_ + 1] * eye64 for p_ in range(4)], axis=0)
    s1[...] = jnp.concatenate(
        [probs[0:1, 4 + p_:5 + p_] * eye64 for p_ in range(4)], axis=0)

    def fold2(g):
        Wp = jnp.dot(W2r[g], L2r[g, :HID, :],
                     preferred_element_type=jnp.float32)
        c = jnp.dot(b2r[g:g + 1, :], L2r[g, :HID, :],
                    preferred_element_type=jnp.float32) + Lb2r[g:g + 1, :]
        return Wp, c

    V0p, d0 = fold2(0)
    V2p, d2 = fold2(2)
    v0big[...] = jnp.concatenate(
        [jnp.concatenate([V0p, z64], 0), jnp.concatenate([z64, V0p], 0)], 1)
    v2big[...] = jnp.concatenate(
        [jnp.concatenate([V2p, z64], 0), jnp.concatenate([z64, V2p], 0)], 1)
    d02[...] = jnp.concatenate([d0] * 2, axis=1)
    d22[...] = jnp.concatenate([d2] * 2, axis=1)
    zlin = jnp.zeros((HID, 1), jnp.float32)
    lw = linwr[...]
    linbig[...] = jnp.concatenate(
        [jnp.concatenate([lw, zlin], 0), jnp.concatenate([zlin, lw], 0)], 1)


def _fold_weights(W1, L1, b1, Lb1, att1, W2, L2, b2, Lb2, lin_w):
    outs = [
        jax.ShapeDtypeStruct((4 * F, 4 * HID), jnp.float32),
        jax.ShapeDtypeStruct((4 * F, 4 * HID), jnp.float32),
        jax.ShapeDtypeStruct((1, 4 * HID), jnp.float32),
        jax.ShapeDtypeStruct((1, 4 * HID), jnp.float32),
        jax.ShapeDtypeStruct((4 * HID, HID), jnp.float32),
        jax.ShapeDtypeStruct((4 * HID, HID), jnp.float32),
        jax.ShapeDtypeStruct((2 * HID, 2 * HID), jnp.float32),
        jax.ShapeDtypeStruct((2 * HID, 2 * HID), jnp.float32),
        jax.ShapeDtypeStruct((1, 2 * HID), jnp.float32),
        jax.ShapeDtypeStruct((1, 2 * HID), jnp.float32),
        jax.ShapeDtypeStruct((2 * HID, 2), jnp.float32),
    ]
    return pl.pallas_call(_foldw_body, out_shape=outs)(
        W1, L1, b1, Lb1, att1[None], W2, L2, b2, Lb2, lin_w)


def _dinv_of(degr):
    dp = degr[...]
    return lax.rsqrt(1.0 + dp[0, :, 0] + dp[1, :, 0])


def _relayout_body(xr, degr, pmr, out):
    dinv = _dinv_of(degr)
    y = jnp.dot(xr[0], pmr[0], preferred_element_type=jnp.float32)
    out[0] = y * dinv[:, None]


def _relayout(x2, deg2, pm):
    grid = (B, 2, NBLK)
    return pl.pallas_call(
        _relayout_body,
        grid=grid,
        in_specs=[
            pl.BlockSpec((1, NB, F * P), lambda b, h, i: (b, i, 0)),
            pl.BlockSpec((2, NB, 1), lambda b, h, i: (0, i, 0)),
            pl.BlockSpec((1, F * P, CH), lambda b, h, i: (h, 0, 0)),
        ],
        out_specs=pl.BlockSpec((1, NB, CH), lambda b, h, i: (b * 2 + h, i, 0)),
        out_shape=jax.ShapeDtypeStruct((2 * B, N, CH), jnp.float32),
    )(x2, deg2, pm)


def _gates1_body(p1r, xcr, degr, w0r, w2r, c04r, c24r, s0r, s1r, out):
    dinv = _dinv_of(degr)[:, None]
    halves = []
    for b_loc in range(2):
        acc = jnp.zeros((NB, HID), jnp.float32)
        for h in range(2):
            kk = b_loc * 2 + h
            M = dinv * (p1r[kk] + xcr[kk])
            z4 = jax.nn.sigmoid(
                jnp.dot(M, w0r[...], preferred_element_type=jnp.float32)
                + c04r[...])
            t4 = jnp.tanh(
                jnp.dot(M, w2r[...], preferred_element_type=jnp.float32)
                + c24r[...])
            w4 = (1.0 - z4) * t4
            sh = s0r[...] if h == 0 else s1r[...]
            acc = acc + jnp.dot(w4, sh, preferred_element_type=jnp.float32)
        halves.append(jax.nn.relu(acc) * dinv)
    out[0] = jnp.concatenate(halves, axis=1)


def _gates1(p1, xc, deg2, w0big, w2big, c04, c24, s0, s1):
    grid = (4, NBLK)
    return pl.pallas_call(
        _gates1_body,
        grid=grid,
        in_specs=[
            pl.BlockSpec((4, NB, CH), lambda j, i: (j, i, 0)),
            pl.BlockSpec((4, NB, CH), lambda j, i: (j, i, 0)),
            pl.BlockSpec((2, NB, 1), lambda j, i: (0, i, 0)),
            pl.BlockSpec((4 * F, 4 * HID), lambda j, i: (0, 0)),
            pl.BlockSpec((4 * F, 4 * HID), lambda j, i: (0, 0)),
            pl.BlockSpec((1, 4 * HID), lambda j, i: (0, 0)),
            pl.BlockSpec((1, 4 * HID), lambda j, i: (0, 0)),
            pl.BlockSpec((4 * HID, HID), lambda j, i: (0, 0)),
            pl.BlockSpec((4 * HID, HID), lambda j, i: (0, 0)),
        ],
        out_specs=pl.BlockSpec((1, NB, CH), lambda j, i: (j, i, 0)),
        out_shape=jax.ShapeDtypeStruct((4, N, CH), jnp.float32),
    )(p1, xc, deg2, w0big, w2big, c04, c24, s0, s1)


def _final_body(p2r, hsr, degr, v0r, v2r, d02r, d22r, linr, lbr, out):
    dinv = _dinv_of(degr)[:, None]
    G = dinv * (p2r[0] + hsr[0])
    z2 = jax.nn.sigmoid(
        jnp.dot(G, v0r[...], preferred_element_type=jnp.float32) + d02r[...])
    t2 = jnp.tanh(
        jnp.dot(G, v2r[...], preferred_element_type=jnp.float32) + d22r[...])
    h2 = jax.nn.relu((1.0 - z2) * t2)
    out[0] = jnp.dot(h2, linr[...], preferred_element_type=jnp.float32) \
        + lbr[0:1, 0:1]


def _final(p2, hs, deg2, v0big, v2big, d02, d22, linbig, lin_b):
    grid = (4, NBLK)
    return pl.pallas_call(
        _final_body,
        grid=grid,
        in_specs=[
            pl.BlockSpec((1, NB, CH), lambda j, i: (j, i, 0)),
            pl.BlockSpec((1, NB, CH), lambda j, i: (j, i, 0)),
            pl.BlockSpec((2, NB, 1), lambda j, i: (0, i, 0)),
            pl.BlockSpec((2 * HID, 2 * HID), lambda j, i: (0, 0)),
            pl.BlockSpec((2 * HID, 2 * HID), lambda j, i: (0, 0)),
            pl.BlockSpec((1, 2 * HID), lambda j, i: (0, 0)),
            pl.BlockSpec((1, 2 * HID), lambda j, i: (0, 0)),
            pl.BlockSpec((2 * HID, 2), lambda j, i: (0, 0)),
            pl.BlockSpec((1, 1), lambda j, i: (0, 0)),
        ],
        out_specs=pl.BlockSpec((1, NB, 2), lambda j, i: (j, i, 0)),
        out_shape=jax.ShapeDtypeStruct((4, N, 2), jnp.float32),
    )(p2, hs, deg2, v0big, v2big, d02, d22, linbig, lin_b)


def kernel(x, edge_index, W1, b1, L1, Lb1, att1, W2, b2, L2, Lb2, att2,
           lin_w, lin_b):
    x2 = x.reshape(B, N, F * P)
    epad = EPAD - E
    pad_block = jnp.stack([
        jnp.zeros((epad,), jnp.int32),
        jnp.full((epad,), NPAD - 1, jnp.int32),
    ])
    er = jnp.concatenate([edge_index, pad_block],
                         axis=1).reshape(2, 2 * NTILE, NBE, EB)
    ones_h = jnp.ones((EB,), jnp.float32)
    zerod_h = jnp.zeros((ROWS_T,), jnp.float32)
    zeros_h = jnp.zeros((ROWS_T, CH), jnp.float32)

    fi = jnp.arange(F * P) // P
    pi = jnp.arange(F * P) % P
    pm = jnp.stack([
        ((pi[:, None] == (h * 4 + jnp.arange(CH)[None, :] // F))
         & (fi[:, None] == jnp.arange(CH)[None, :] % F)).astype(jnp.float32)
        for h in range(2)
    ])

    (w0big, w2big, c04, c24, s0, s1, v0big, v2big, d02, d22, linbig) = \
        _fold_weights(W1, L1, b1, Lb1, att1, W2, L2, b2, Lb2, lin_w)

    deg_p = _deg_kernel(er, ones_h, zerod_h)
    deg2 = deg_p[:, :, None]
    xc = _relayout(x2, deg2, pm)
    p1 = _prop16(xc, er, zeros_h)
    hs = _gates1(p1, xc, deg2, w0big, w2big, c04, c24, s0, s1)
    p2 = _prop4(hs, er, zeros_h)
    out3 = _final(p2, hs, deg2, v0big, v2big, d02, d22, linbig,
                  lin_b.reshape(1, 1))
    return out3.transpose(0, 2, 1).reshape(B, N)


# speedup vs baseline: 160.3060x; 1.2109x over previous
import functools
import jax
import jax.numpy as jnp
from jax import lax
from jax.experimental import pallas as pl
from jax.experimental.pallas import tpu as pltpu
from jax.experimental.pallas import tpu_sc as plsc

N = 10000
E = 160000
B = 8
F = 32
P = 8
HID = 64
CH = 128
EB = 128
EPAD = 163840
NBE = 40
DW = 8
NWIN = 2 * NBE // DW
NTILE = 16
NPAD = 10240
ROWS_T = NPAD // NTILE
NB = 2000
NBLK = N // NB

_mesh = plsc.VectorSubcoreMesh(core_axis_name="c", subcore_axis_name="s")


@functools.partial(
    pl.kernel,
    out_type=jax.ShapeDtypeStruct((2, NPAD), jnp.float32),
    mesh=_mesh,
    scratch_types=[
        pltpu.VMEM((NBE, EB), jnp.int32),
        pltpu.VMEM((EB,), jnp.float32),
        pltpu.VMEM_SHARED((NPAD,), jnp.float32),
    ],
)
def _deg_kernel(er, ones_h, zeros_h, deg_out, dst_v, ones_v, acc):
    cid = lax.axis_index("c")
    sid = lax.axis_index("s")
    pltpu.sync_copy(er.at[1, cid * NTILE + sid], dst_v)
    pltpu.sync_copy(ones_h, ones_v)
    pltpu.sync_copy(zeros_h, acc.at[pl.ds(sid * ROWS_T, ROWS_T)])
    plsc.subcore_barrier()

    def eb_body(eb, carry):
        pltpu.sync_copy(ones_v, acc.at[dst_v.at[eb]], add=True)
        return carry

    lax.fori_loop(0, NBE, eb_body, 0)
    plsc.subcore_barrier()
    pltpu.sync_copy(
        acc.at[pl.ds(sid * ROWS_T, ROWS_T)],
        deg_out.at[cid, pl.ds(sid * ROWS_T, ROWS_T)],
    )


def _make_prop(nchunk):
    nck = nchunk // 2

    @functools.partial(
        pl.kernel,
        out_type=jax.ShapeDtypeStruct((nchunk, NPAD, CH), jnp.float32),
        mesh=_mesh,
        scratch_types=[
            pltpu.VMEM((2 * NBE, EB), jnp.int32),
            pltpu.VMEM((2 * DW, EB), jnp.int32),
            pltpu.VMEM((EB, CH), jnp.float32),
            pltpu.VMEM((EB, CH), jnp.float32),
            pltpu.VMEM_SHARED((NPAD, CH), jnp.float32),
            pltpu.SemaphoreType.DMA,
            pltpu.SemaphoreType.DMA,
            pltpu.SemaphoreType.DMA,
            pltpu.SemaphoreType.DMA,
        ],
    )
    def prop(xc, er, zeros_h, out, src_v, dstw, rows_a, rows_b, acc,
             sem_a, sem_b, sem_d0, sem_d1):
        cid = lax.axis_index("c")
        sid = lax.axis_index("s")
        pltpu.sync_copy(er.at[0, sid], src_v.at[pl.ds(0, NBE)])
        pltpu.sync_copy(er.at[0, sid + NTILE], src_v.at[pl.ds(NBE, NBE)])

        def dstw_desc(wt, half, sem):
            blk = sid + jnp.where(wt >= NBE // DW, NTILE, 0)
            r0 = DW * wt - jnp.where(wt >= NBE // DW, NBE, 0)
            return pltpu.make_async_copy(
                er.at[1, blk, pl.ds(r0, DW)],
                dstw.at[pl.ds(half * DW, DW)], sem)

        def chunk_body(ci, carry):
            chunk = cid * nck + ci
            pltpu.sync_copy(zeros_h, acc.at[pl.ds(sid * ROWS_T, ROWS_T)])
            plsc.subcore_barrier()

            dstw_desc(0, 0, sem_d0).start()
            dstw_desc(1, 1, sem_d1).start()
            pltpu.async_copy(xc.at[chunk].at[src_v.at[0]], rows_a, sem_a)

            def window(w, carry):
                half = lax.rem(w, 2)
                hb = half * DW

                @pl.when(half == 0)
                def _():
                    dstw_desc(w, 0, sem_d0).wait()

                @pl.when(half == 1)
                def _():
                    dstw_desc(w, 1, sem_d1).wait()

                for jp in range(DW // 2):
                    eb = DW * w + 2 * jp
                    nxt = eb + 1
                    pltpu.async_copy(xc.at[chunk].at[src_v.at[nxt]], rows_b,
                                     sem_b)
                    pltpu.make_async_copy(xc.at[chunk].at[src_v.at[eb]],
                                          rows_a, sem_a).wait()
                    pltpu.sync_copy(rows_a, acc.at[dstw.at[hb + 2 * jp]],
                                    add=True)

                    nxt2 = eb + 2

                    @pl.when(nxt2 < 2 * NBE)
                    def _():
                        pltpu.async_copy(xc.at[chunk].at[src_v.at[nxt2]],
                                         rows_a, sem_a)

                    pltpu.make_async_copy(xc.at[chunk].at[src_v.at[nxt]],
                                          rows_b, sem_b).wait()
                    pltpu.sync_copy(rows_b, acc.at[dstw.at[hb + 2 * jp + 1]],
                                    add=True)

                nxtw = w + 2

                @pl.when((nxtw < NWIN) & (half == 0))
                def _():
                    dstw_desc(nxtw, 0, sem_d0).start()

                @pl.when((nxtw < NWIN) & (half == 1))
                def _():
                    dstw_desc(nxtw, 1, sem_d1).start()

                return carry

            lax.fori_loop(0, NWIN, window, 0)
            plsc.subcore_barrier()
            pltpu.sync_copy(
                acc.at[pl.ds(sid * ROWS_T, ROWS_T)],
                out.at[chunk, pl.ds(sid * ROWS_T, ROWS_T)],
            )
            plsc.subcore_barrier()
            return carry

        lax.fori_loop(0, nck, chunk_body, 0)

    return prop


_prop16 = _make_prop(16)
_prop4 = _make_prop(4)


def _foldw_body(W1r, L1r, b1r, Lb1r, att1r, W2r, L2r, b2r, Lb2r, linwr,
                w0big, w2big, c04, c24, s0, s1, v0big, v2big, d02, d22,
                linbig):
    z3264 = jnp.zeros((F, HID), jnp.float32)
    z64 = jnp.zeros((HID, HID), jnp.float32)
    eye64 = jnp.eye(HID, dtype=jnp.float32)

    def fold1(g):
        Wp = jnp.dot(W1r[g], L1r[g, :HID, :],
                     preferred_element_type=jnp.float32)
        c = jnp.dot(b1r[g:g + 1, :], L1r[g, :HID, :],
                    preferred_element_type=jnp.float32) + Lb1r[g:g + 1, :]
        return Wp, c

    W0p, c0 = fold1(0)
    W2p, c2 = fold1(2)

    def bigdiag4(Wp):
        cols = []
        for j in range(4):
            blocks = [Wp if i == j else z3264 for i in range(4)]
            cols.append(jnp.concatenate(blocks, axis=0))
        return jnp.concatenate(cols, axis=1)

    w0big[...] = bigdiag4(W0p)
    w2big[...] = bigdiag4(W2p)
    c04[...] = jnp.concatenate([c0] * 4, axis=1)
    c24[...] = jnp.concatenate([c2] * 4, axis=1)

    probs = jax.nn.softmax(att1r[...], axis=-1)
    s0[...] = jnp.concatenate(
        [probs[0:1, p_:p_ + 1] * eye64 for p_ in range(4)], axis=0)
    s1[...] = jnp.concatenate(
        [probs[0:1, 4 + p_:5 + p_] * eye64 for p_ in range(4)], axis=0)

    def fold2(g):
        Wp = jnp.dot(W2r[g], L2r[g, :HID, :],
                     preferred_element_type=jnp.float32)
        c = jnp.dot(b2r[g:g + 1, :], L2r[g, :HID, :],
                    preferred_element_type=jnp.float32) + Lb2r[g:g + 1, :]
        return Wp, c

    V0p, d0 = fold2(0)
    V2p, d2 = fold2(2)
    v0big[...] = jnp.concatenate(
        [jnp.concatenate([V0p, z64], 0), jnp.concatenate([z64, V0p], 0)], 1)
    v2big[...] = jnp.concatenate(
        [jnp.concatenate([V2p, z64], 0), jnp.concatenate([z64, V2p], 0)], 1)
    d02[...] = jnp.concatenate([d0] * 2, axis=1)
    d22[...] = jnp.concatenate([d2] * 2, axis=1)
    zlin = jnp.zeros((HID, 1), jnp.float32)
    lw = linwr[...]
    linbig[...] = jnp.concatenate(
        [jnp.concatenate([lw, zlin], 0), jnp.concatenate([zlin, lw], 0)], 1)


def _fold_weights(W1, L1, b1, Lb1, att1, W2, L2, b2, Lb2, lin_w):
    outs = [
        jax.ShapeDtypeStruct((4 * F, 4 * HID), jnp.float32),
        jax.ShapeDtypeStruct((4 * F, 4 * HID), jnp.float32),
        jax.ShapeDtypeStruct((1, 4 * HID), jnp.float32),
        jax.ShapeDtypeStruct((1, 4 * HID), jnp.float32),
        jax.ShapeDtypeStruct((4 * HID, HID), jnp.float32),
        jax.ShapeDtypeStruct((4 * HID, HID), jnp.float32),
        jax.ShapeDtypeStruct((2 * HID, 2 * HID), jnp.float32),
        jax.ShapeDtypeStruct((2 * HID, 2 * HID), jnp.float32),
        jax.ShapeDtypeStruct((1, 2 * HID), jnp.float32),
        jax.ShapeDtypeStruct((1, 2 * HID), jnp.float32),
        jax.ShapeDtypeStruct((2 * HID, 2), jnp.float32),
    ]
    return pl.pallas_call(_foldw_body, out_shape=outs)(
        W1, L1, b1, Lb1, att1[None], W2, L2, b2, Lb2, lin_w)


def _dinv_of(degr):
    dp = degr[...]
    return lax.rsqrt(1.0 + dp[0, :, 0] + dp[1, :, 0])


def _relayout_body(xr, degr, pmr, out):
    dinv = _dinv_of(degr)
    y = jnp.dot(xr[0], pmr[0], preferred_element_type=jnp.float32)
    out[0] = y * dinv[:, None]


def _relayout(x2, deg2, pm):
    grid = (B, 2, NBLK)
    return pl.pallas_call(
        _relayout_body,
        grid=grid,
        in_specs=[
            pl.BlockSpec((1, NB, F * P), lambda b, h, i: (b, i, 0)),
            pl.BlockSpec((2, NB, 1), lambda b, h, i: (0, i, 0)),
            pl.BlockSpec((1, F * P, CH), lambda b, h, i: (h, 0, 0)),
        ],
        out_specs=pl.BlockSpec((1, NB, CH), lambda b, h, i: (b * 2 + h, i, 0)),
        out_shape=jax.ShapeDtypeStruct((2 * B, N, CH), jnp.float32),
    )(x2, deg2, pm)


def _gates1_body(p1r, xcr, degr, w0r, w2r, c04r, c24r, s0r, s1r, out):
    dinv = _dinv_of(degr)[:, None]
    halves = []
    for b_loc in range(2):
        acc = jnp.zeros((NB, HID), jnp.float32)
        for h in range(2):
            kk = b_loc * 2 + h
            M = dinv * (p1r[kk] + xcr[kk])
            z4 = jax.nn.sigmoid(
                jnp.dot(M, w0r[...], preferred_element_type=jnp.float32)
                + c04r[...])
            t4 = jnp.tanh(
                jnp.dot(M, w2r[...], preferred_element_type=jnp.float32)
                + c24r[...])
            w4 = (1.0 - z4) * t4
            sh = s0r[...] if h == 0 else s1r[...]
            acc = acc + jnp.dot(w4, sh, preferred_element_type=jnp.float32)
        halves.append(jax.nn.relu(acc) * dinv)
    out[0] = jnp.concatenate(halves, axis=1)


def _gates1(p1, xc, deg2, w0big, w2big, c04, c24, s0, s1):
    grid = (4, NBLK)
    return pl.pallas_call(
        _gates1_body,
        grid=grid,
        in_specs=[
            pl.BlockSpec((4, NB, CH), lambda j, i: (j, i, 0)),
            pl.BlockSpec((4, NB, CH), lambda j, i: (j, i, 0)),
            pl.BlockSpec((2, NB, 1), lambda j, i: (0, i, 0)),
            pl.BlockSpec((4 * F, 4 * HID), lambda j, i: (0, 0)),
            pl.BlockSpec((4 * F, 4 * HID), lambda j, i: (0, 0)),
            pl.BlockSpec((1, 4 * HID), lambda j, i: (0, 0)),
            pl.BlockSpec((1, 4 * HID), lambda j, i: (0, 0)),
            pl.BlockSpec((4 * HID, HID), lambda j, i: (0, 0)),
            pl.BlockSpec((4 * HID, HID), lambda j, i: (0, 0)),
        ],
        out_specs=pl.BlockSpec((1, NB, CH), lambda j, i: (j, i, 0)),
        out_shape=jax.ShapeDtypeStruct((4, N, CH), jnp.float32),
    )(p1, xc, deg2, w0big, w2big, c04, c24, s0, s1)


def _final_body(p2r, hsr, degr, v0r, v2r, d02r, d22r, linr, lbr, out):
    dinv = _dinv_of(degr)[:, None]
    G = dinv * (p2r[0] + hsr[0])
    z2 = jax.nn.sigmoid(
        jnp.dot(G, v0r[...], preferred_element_type=jnp.float32) + d02r[...])
    t2 = jnp.tanh(
        jnp.dot(G, v2r[...], preferred_element_type=jnp.float32) + d22r[...])
    h2 = jax.nn.relu((1.0 - z2) * t2)
    out[0] = jnp.dot(h2, linr[...], preferred_element_type=jnp.float32) \
        + lbr[0:1, 0:1]


def _final(p2, hs, deg2, v0big, v2big, d02, d22, linbig, lin_b):
    grid = (4, NBLK)
    return pl.pallas_call(
        _final_body,
        grid=grid,
        in_specs=[
            pl.BlockSpec((1, NB, CH), lambda j, i: (j, i, 0)),
            pl.BlockSpec((1, NB, CH), lambda j, i: (j, i, 0)),
            pl.BlockSpec((2, NB, 1), lambda j, i: (0, i, 0)),
            pl.BlockSpec((2 * HID, 2 * HID), lambda j, i: (0, 0)),
            pl.BlockSpec((2 * HID, 2 * HID), lambda j, i: (0, 0)),
            pl.BlockSpec((1, 2 * HID), lambda j, i: (0, 0)),
            pl.BlockSpec((1, 2 * HID), lambda j, i: (0, 0)),
            pl.BlockSpec((2 * HID, 2), lambda j, i: (0, 0)),
            pl.BlockSpec((1, 1), lambda j, i: (0, 0)),
        ],
        out_specs=pl.BlockSpec((1, NB, 2), lambda j, i: (j, i, 0)),
        out_shape=jax.ShapeDtypeStruct((4, N, 2), jnp.float32),
    )(p2, hs, deg2, v0big, v2big, d02, d22, linbig, lin_b)


def kernel(x, edge_index, W1, b1, L1, Lb1, att1, W2, b2, L2, Lb2, att2,
           lin_w, lin_b):
    x2 = x.reshape(B, N, F * P)
    epad = EPAD - E
    pad_block = jnp.stack([
        jnp.zeros((epad,), jnp.int32),
        jnp.full((epad,), NPAD - 1, jnp.int32),
    ])
    er = jnp.concatenate([edge_index, pad_block],
                         axis=1).reshape(2, 2 * NTILE, NBE, EB)
    ones_h = jnp.ones((EB,), jnp.float32)
    zerod_h = jnp.zeros((ROWS_T,), jnp.float32)
    zeros_h = jnp.zeros((ROWS_T, CH), jnp.float32)

    fi = jnp.arange(F * P) // P
    pi = jnp.arange(F * P) % P
    pm = jnp.stack([
        ((pi[:, None] == (h * 4 + jnp.arange(CH)[None, :] // F))
         & (fi[:, None] == jnp.arange(CH)[None, :] % F)).astype(jnp.float32)
        for h in range(2)
    ])

    (w0big, w2big, c04, c24, s0, s1, v0big, v2big, d02, d22, linbig) = \
        _fold_weights(W1, L1, b1, Lb1, att1, W2, L2, b2, Lb2, lin_w)

    deg_p = _deg_kernel(er, ones_h, zerod_h)
    deg2 = deg_p[:, :, None]
    xc = _relayout(x2, deg2, pm)
    p1 = _prop16(xc, er, zeros_h)
    hs = _gates1(p1, xc, deg2, w0big, w2big, c04, c24, s0, s1)
    p2 = _prop4(hs, er, zeros_h)
    out3 = _final(p2, hs, deg2, v0big, v2big, d02, d22, linbig,
                  lin_b.reshape(1, 1))
    return out3.transpose(0, 2, 1).reshape(B, N)


# speedup vs baseline: 160.3518x; 1.0003x over previous
import functools
import jax
import jax.numpy as jnp
from jax import lax
from jax.experimental import pallas as pl
from jax.experimental.pallas import tpu as pltpu
from jax.experimental.pallas import tpu_sc as plsc

N = 10000
E = 160000
B = 8
F = 32
P = 8
HID = 64
CH = 128
EB = 128
EPAD = 163840
NBE = 40
DW = 8
NWIN = 2 * NBE // DW
NTILE = 16
NPAD = 10240
ROWS_T = NPAD // NTILE
NB = 2000
NBLK = N // NB

_mesh = plsc.VectorSubcoreMesh(core_axis_name="c", subcore_axis_name="s")


@functools.partial(
    pl.kernel,
    out_type=jax.ShapeDtypeStruct((2, NPAD), jnp.float32),
    mesh=_mesh,
    scratch_types=[
        pltpu.VMEM((NBE, EB), jnp.int32),
        pltpu.VMEM((EB,), jnp.float32),
        pltpu.VMEM_SHARED((NPAD,), jnp.float32),
    ],
)
def _deg_kernel(er, ones_h, zeros_h, deg_out, dst_v, ones_v, acc):
    cid = lax.axis_index("c")
    sid = lax.axis_index("s")
    pltpu.sync_copy(er.at[1, cid * NTILE + sid], dst_v)
    pltpu.sync_copy(ones_h, ones_v)
    pltpu.sync_copy(zeros_h, acc.at[pl.ds(sid * ROWS_T, ROWS_T)])
    plsc.subcore_barrier()

    def eb_body(eb, carry):
        pltpu.sync_copy(ones_v, acc.at[dst_v.at[eb]], add=True)
        return carry

    lax.fori_loop(0, NBE, eb_body, 0)
    plsc.subcore_barrier()
    pltpu.sync_copy(
        acc.at[pl.ds(sid * ROWS_T, ROWS_T)],
        deg_out.at[cid, pl.ds(sid * ROWS_T, ROWS_T)],
    )


def _make_prop(nchunk):
    nck = nchunk // 2

    @functools.partial(
        pl.kernel,
        out_type=jax.ShapeDtypeStruct((nchunk, NPAD, CH), jnp.float32),
        mesh=_mesh,
        scratch_types=[
            pltpu.VMEM((2 * NBE, EB), jnp.int32),
            pltpu.VMEM((2 * DW, EB), jnp.int32),
            pltpu.VMEM((EB, CH), jnp.float32),
            pltpu.VMEM((EB, CH), jnp.float32),
            pltpu.VMEM_SHARED((NPAD, CH), jnp.float32),
            pltpu.SemaphoreType.DMA,
            pltpu.SemaphoreType.DMA,
            pltpu.SemaphoreType.DMA,
            pltpu.SemaphoreType.DMA,
        ],
    )
    def prop(xc, er, zeros_h, out, src_v, dstw, rows_a, rows_b, acc,
             sem_a, sem_b, sem_d0, sem_d1):
        cid = lax.axis_index("c")
        sid = lax.axis_index("s")
        pltpu.sync_copy(er.at[0, sid], src_v.at[pl.ds(0, NBE)])
        pltpu.sync_copy(er.at[0, sid + NTILE], src_v.at[pl.ds(NBE, NBE)])

        def dstw_desc(wt, half, sem):
            blk = sid + jnp.where(wt >= NBE // DW, NTILE, 0)
            r0 = DW * wt - jnp.where(wt >= NBE // DW, NBE, 0)
            return pltpu.make_async_copy(
                er.at[1, blk, pl.ds(r0, DW)],
                dstw.at[pl.ds(half * DW, DW)], sem)

        def gat(eb, buf, sem):
            pltpu.async_copy(xc.at[chunk_ref[0]].at[src_v.at[eb, pl.ds(0, EB // 2)]],
                             buf.at[pl.ds(0, EB // 2)], sem)
            pltpu.async_copy(xc.at[chunk_ref[0]].at[src_v.at[eb, pl.ds(EB // 2, EB // 2)]],
                             buf.at[pl.ds(EB // 2, EB // 2)], sem)

        def gwait(eb, buf, sem):
            pltpu.make_async_copy(xc.at[chunk_ref[0]].at[src_v.at[eb]], buf,
                                  sem).wait()

        chunk_ref = [None]

        def chunk_body(ci, carry):
            chunk = cid * nck + ci
            chunk_ref[0] = chunk
            pltpu.sync_copy(zeros_h, acc.at[pl.ds(sid * ROWS_T, ROWS_T)])
            plsc.subcore_barrier()

            dstw_desc(0, 0, sem_d0).start()
            dstw_desc(1, 1, sem_d1).start()
            gat(0, rows_a, sem_a)

            def window(w, carry):
                half = lax.rem(w, 2)
                hb = half * DW

                @pl.when(half == 0)
                def _():
                    dstw_desc(w, 0, sem_d0).wait()

                @pl.when(half == 1)
                def _():
                    dstw_desc(w, 1, sem_d1).wait()

                for jp in range(DW // 2):
                    eb = DW * w + 2 * jp
                    nxt = eb + 1
                    gat(nxt, rows_b, sem_b)
                    gwait(eb, rows_a, sem_a)
                    pltpu.sync_copy(rows_a, acc.at[dstw.at[hb + 2 * jp]],
                                    add=True)

                    nxt2 = eb + 2

                    @pl.when(nxt2 < 2 * NBE)
                    def _():
                        gat(nxt2, rows_a, sem_a)

                    gwait(nxt, rows_b, sem_b)
                    pltpu.sync_copy(rows_b, acc.at[dstw.at[hb + 2 * jp + 1]],
                                    add=True)

                nxtw = w + 2

                @pl.when((nxtw < NWIN) & (half == 0))
                def _():
                    dstw_desc(nxtw, 0, sem_d0).start()

                @pl.when((nxtw < NWIN) & (half == 1))
                def _():
                    dstw_desc(nxtw, 1, sem_d1).start()

                return carry

            lax.fori_loop(0, NWIN, window, 0)
            plsc.subcore_barrier()
            pltpu.sync_copy(
                acc.at[pl.ds(sid * ROWS_T, ROWS_T)],
                out.at[chunk, pl.ds(sid * ROWS_T, ROWS_T)],
            )
            plsc.subcore_barrier()
            return carry

        lax.fori_loop(0, nck, chunk_body, 0)

    return prop


_prop16 = _make_prop(16)
_prop4 = _make_prop(4)


def _foldw_body(W1r, L1r, b1r, Lb1r, att1r, W2r, L2r, b2r, Lb2r, linwr,
                w0big, w2big, c04, c24, s0, s1, v0big, v2big, d02, d22,
                linbig):
    z3264 = jnp.zeros((F, HID), jnp.float32)
    z64 = jnp.zeros((HID, HID), jnp.float32)
    eye64 = jnp.eye(HID, dtype=jnp.float32)

    def fold1(g):
        Wp = jnp.dot(W1r[g], L1r[g, :HID, :],
                     preferred_element_type=jnp.float32)
        c = jnp.dot(b1r[g:g + 1, :], L1r[g, :HID, :],
                    preferred_element_type=jnp.float32) + Lb1r[g:g + 1, :]
        return Wp, c

    W0p, c0 = fold1(0)
    W2p, c2 = fold1(2)

    def bigdiag4(Wp):
        cols = []
        for j in range(4):
            blocks = [Wp if i == j else z3264 for i in range(4)]
            cols.append(jnp.concatenate(blocks, axis=0))
        return jnp.concatenate(cols, axis=1)

    w0big[...] = bigdiag4(W0p)
    w2big[...] = bigdiag4(W2p)
    c04[...] = jnp.concatenate([c0] * 4, axis=1)
    c24[...] = jnp.concatenate([c2] * 4, axis=1)

    probs = jax.nn.softmax(att1r[...], axis=-1)
    s0[...] = jnp.concatenate(
        [probs[0:1, p_:p_ + 1] * eye64 for p_ in range(4)], axis=0)
    s1[...] = jnp.concatenate(
        [probs[0:1, 4 + p_:5 + p_] * eye64 for p_ in range(4)], axis=0)

    def fold2(g):
        Wp = jnp.dot(W2r[g], L2r[g, :HID, :],
                     preferred_element_type=jnp.float32)
        c = jnp.dot(b2r[g:g + 1, :], L2r[g, :HID, :],
                    preferred_element_type=jnp.float32) + Lb2r[g:g + 1, :]
        return Wp, c

    V0p, d0 = fold2(0)
    V2p, d2 = fold2(2)
    v0big[...] = jnp.concatenate(
        [jnp.concatenate([V0p, z64], 0), jnp.concatenate([z64, V0p], 0)], 1)
    v2big[...] = jnp.concatenate(
        [jnp.concatenate([V2p, z64], 0), jnp.concatenate([z64, V2p], 0)], 1)
    d02[...] = jnp.concatenate([d0] * 2, axis=1)
    d22[...] = jnp.concatenate([d2] * 2, axis=1)
    zlin = jnp.zeros((HID, 1), jnp.float32)
    lw = linwr[...]
    linbig[...] = jnp.concatenate(
        [jnp.concatenate([lw, zlin], 0), jnp.concatenate([zlin, lw], 0)], 1)


def _fold_weights(W1, L1, b1, Lb1, att1, W2, L2, b2, Lb2, lin_w):
    outs = [
        jax.ShapeDtypeStruct((4 * F, 4 * HID), jnp.float32),
        jax.ShapeDtypeStruct((4 * F, 4 * HID), jnp.float32),
        jax.ShapeDtypeStruct((1, 4 * HID), jnp.float32),
        jax.ShapeDtypeStruct((1, 4 * HID), jnp.float32),
        jax.ShapeDtypeStruct((4 * HID, HID), jnp.float32),
        jax.ShapeDtypeStruct((4 * HID, HID), jnp.float32),
        jax.ShapeDtypeStruct((2 * HID, 2 * HID), jnp.float32),
        jax.ShapeDtypeStruct((2 * HID, 2 * HID), jnp.float32),
        jax.ShapeDtypeStruct((1, 2 * HID), jnp.float32),
        jax.ShapeDtypeStruct((1, 2 * HID), jnp.float32),
        jax.ShapeDtypeStruct((2 * HID, 2), jnp.float32),
    ]
    return pl.pallas_call(_foldw_body, out_shape=outs)(
        W1, L1, b1, Lb1, att1[None], W2, L2, b2, Lb2, lin_w)


def _dinv_of(degr):
    dp = degr[...]
    return lax.rsqrt(1.0 + dp[0, :, 0] + dp[1, :, 0])


def _relayout_body(xr, degr, pmr, out):
    dinv = _dinv_of(degr)
    y = jnp.dot(xr[0], pmr[0], preferred_element_type=jnp.float32)
    out[0] = y * dinv[:, None]


def _relayout(x2, deg2, pm):
    grid = (B, 2, NBLK)
    return pl.pallas_call(
        _relayout_body,
        grid=grid,
        in_specs=[
            pl.BlockSpec((1, NB, F * P), lambda b, h, i: (b, i, 0)),
            pl.BlockSpec((2, NB, 1), lambda b, h, i: (0, i, 0)),
            pl.BlockSpec((1, F * P, CH), lambda b, h, i: (h, 0, 0)),
        ],
        out_specs=pl.BlockSpec((1, NB, CH), lambda b, h, i: (b * 2 + h, i, 0)),
        out_shape=jax.ShapeDtypeStruct((2 * B, N, CH), jnp.float32),
    )(x2, deg2, pm)


def _gates1_body(p1r, xcr, degr, w0r, w2r, c04r, c24r, s0r, s1r, out):
    dinv = _dinv_of(degr)[:, None]
    halves = []
    for b_loc in range(2):
        acc = jnp.zeros((NB, HID), jnp.float32)
        for h in range(2):
            kk = b_loc * 2 + h
            M = dinv * (p1r[kk] + xcr[kk])
            z4 = jax.nn.sigmoid(
                jnp.dot(M, w0r[...], preferred_element_type=jnp.float32)
                + c04r[...])
            t4 = jnp.tanh(
                jnp.dot(M, w2r[...], preferred_element_type=jnp.float32)
                + c24r[...])
            w4 = (1.0 - z4) * t4
            sh = s0r[...] if h == 0 else s1r[...]
            acc = acc + jnp.dot(w4, sh, preferred_element_type=jnp.float32)
        halves.append(jax.nn.relu(acc) * dinv)
    out[0] = jnp.concatenate(halves, axis=1)


def _gates1(p1, xc, deg2, w0big, w2big, c04, c24, s0, s1):
    grid = (4, NBLK)
    return pl.pallas_call(
        _gates1_body,
        grid=grid,
        in_specs=[
            pl.BlockSpec((4, NB, CH), lambda j, i: (j, i, 0)),
            pl.BlockSpec((4, NB, CH), lambda j, i: (j, i, 0)),
            pl.BlockSpec((2, NB, 1), lambda j, i: (0, i, 0)),
            pl.BlockSpec((4 * F, 4 * HID), lambda j, i: (0, 0)),
            pl.BlockSpec((4 * F, 4 * HID), lambda j, i: (0, 0)),
            pl.BlockSpec((1, 4 * HID), lambda j, i: (0, 0)),
            pl.BlockSpec((1, 4 * HID), lambda j, i: (0, 0)),
            pl.BlockSpec((4 * HID, HID), lambda j, i: (0, 0)),
            pl.BlockSpec((4 * HID, HID), lambda j, i: (0, 0)),
        ],
        out_specs=pl.BlockSpec((1, NB, CH), lambda j, i: (j, i, 0)),
        out_shape=jax.ShapeDtypeStruct((4, N, CH), jnp.float32),
    )(p1, xc, deg2, w0big, w2big, c04, c24, s0, s1)


def _final_body(p2r, hsr, degr, v0r, v2r, d02r, d22r, linr, lbr, out):
    dinv = _dinv_of(degr)[:, None]
    G = dinv * (p2r[0] + hsr[0])
    z2 = jax.nn.sigmoid(
        jnp.dot(G, v0r[...], preferred_element_type=jnp.float32) + d02r[...])
    t2 = jnp.tanh(
        jnp.dot(G, v2r[...], preferred_element_type=jnp.float32) + d22r[...])
    h2 = jax.nn.relu((1.0 - z2) * t2)
    out[0] = jnp.dot(h2, linr[...], preferred_element_type=jnp.float32) \
        + lbr[0:1, 0:1]


def _final(p2, hs, deg2, v0big, v2big, d02, d22, linbig, lin_b):
    grid = (4, NBLK)
    return pl.pallas_call(
        _final_body,
        grid=grid,
        in_specs=[
            pl.BlockSpec((1, NB, CH), lambda j, i: (j, i, 0)),
            pl.BlockSpec((1, NB, CH), lambda j, i: (j, i, 0)),
            pl.BlockSpec((2, NB, 1), lambda j, i: (0, i, 0)),
            pl.BlockSpec((2 * HID, 2 * HID), lambda j, i: (0, 0)),
            pl.BlockSpec((2 * HID, 2 * HID), lambda j, i: (0, 0)),
            pl.BlockSpec((1, 2 * HID), lambda j, i: (0, 0)),
            pl.BlockSpec((1, 2 * HID), lambda j, i: (0, 0)),
            pl.BlockSpec((2 * HID, 2), lambda j, i: (0, 0)),
            pl.BlockSpec((1, 1), lambda j, i: (0, 0)),
        ],
        out_specs=pl.BlockSpec((1, NB, 2), lambda j, i: (j, i, 0)),
        out_shape=jax.ShapeDtypeStruct((4, N, 2), jnp.float32),
    )(p2, hs, deg2, v0big, v2big, d02, d22, linbig, lin_b)


def kernel(x, edge_index, W1, b1, L1, Lb1, att1, W2, b2, L2, Lb2, att2,
           lin_w, lin_b):
    x2 = x.reshape(B, N, F * P)
    epad = EPAD - E
    pad_block = jnp.stack([
        jnp.zeros((epad,), jnp.int32),
        jnp.full((epad,), NPAD - 1, jnp.int32),
    ])
    er = jnp.concatenate([edge_index, pad_block],
                         axis=1).reshape(2, 2 * NTILE, NBE, EB)
    ones_h = jnp.ones((EB,), jnp.float32)
    zerod_h = jnp.zeros((ROWS_T,), jnp.float32)
    zeros_h = jnp.zeros((ROWS_T, CH), jnp.float32)

    fi = jnp.arange(F * P) // P
    pi = jnp.arange(F * P) % P
    pm = jnp.stack([
        ((pi[:, None] == (h * 4 + jnp.arange(CH)[None, :] // F))
         & (fi[:, None] == jnp.arange(CH)[None, :] % F)).astype(jnp.float32)
        for h in range(2)
    ])

    (w0big, w2big, c04, c24, s0, s1, v0big, v2big, d02, d22, linbig) = \
        _fold_weights(W1, L1, b1, Lb1, att1, W2, L2, b2, Lb2, lin_w)

    deg_p = _deg_kernel(er, ones_h, zerod_h)
    deg2 = deg_p[:, :, None]
    xc = _relayout(x2, deg2, pm)
    p1 = _prop16(xc, er, zeros_h)
    hs = _gates1(p1, xc, deg2, w0big, w2big, c04, c24, s0, s1)
    p2 = _prop4(hs, er, zeros_h)
    out3 = _final(p2, hs, deg2, v0big, v2big, d02, d22, linbig,
                  lin_b.reshape(1, 1))
    return out3.transpose(0, 2, 1).reshape(B, N)
